# baseline (speedup 1.0000x reference)
import functools

import jax
import jax.numpy as jnp
from jax import lax
from jax.experimental import pallas as pl
from jax.experimental.pallas import tpu as pltpu
from jax.experimental.pallas import tpu_sc as plsc

N = 10000
E = 160000
HALF = E // 2
D = 100
DP = 128
H2 = 64
NTILES = 32
NSUB = 16
EPT = E // NTILES
K = 40
CH = EPT // K
RPS = N // NSUB
KC = 25
BN = 400
NB = N // BN
BE = 800
NBE = HALF // BE
VL = 16


def _pad_w(w):
    r, c = w.shape
    return jnp.pad(w, ((0, DP - r), (0, DP - c)))


def _pad_b(b):
    return jnp.pad(b, (0, DP - b.shape[0]))[None, :]


def _proj_body(x_ref, ws_ref, wd_ref, ps_ref, pd_ref):
    x = x_ref[...]
    ps_ref[0] = jnp.dot(x, ws_ref[0], preferred_element_type=jnp.float32)
    pd_ref[0] = jnp.dot(x, wd_ref[0], preferred_element_type=jnp.float32)


def _node_proj(x_pad, ws, wd):
    return pl.pallas_call(
        _proj_body,
        grid=(2, NB),
        in_specs=[
            pl.BlockSpec((BN, DP), lambda h, i: (i, 0)),
            pl.BlockSpec((1, DP, DP), lambda h, i: (h, 0, 0)),
            pl.BlockSpec((1, DP, DP), lambda h, i: (h, 0, 0)),
        ],
        out_specs=[
            pl.BlockSpec((1, BN, DP), lambda h, i: (h, i, 0)),
            pl.BlockSpec((1, BN, DP), lambda h, i: (h, i, 0)),
        ],
        out_shape=[
            jax.ShapeDtypeStruct((2, N, DP), jnp.float32),
            jax.ShapeDtypeStruct((2, N, DP), jnp.float32),
        ],
    )(x_pad, ws, wd)


def _ea_body(ea_ref, w1_ref, b1_ref, w2_ref, b2_ref, o1_ref, o2_ref):
    ea = ea_ref[...]
    o1_ref[...] = jnp.dot(ea, w1_ref[0], preferred_element_type=jnp.float32) + b1_ref[0]
    o2_ref[...] = jnp.dot(ea, w2_ref[0], preferred_element_type=jnp.float32) + b2_ref[0]


def _ea_proj(ea, w1, b1, w2, b2):
    return pl.pallas_call(
        _ea_body,
        grid=(2, NBE),
        in_specs=[
            pl.BlockSpec((BE, D), lambda h, i: (h * NBE + i, 0)),
            pl.BlockSpec((1, D, DP), lambda h, i: (h, 0, 0)),
            pl.BlockSpec((1, 1, DP), lambda h, i: (h, 0, 0)),
            pl.BlockSpec((1, D, DP), lambda h, i: (h, 0, 0)),
            pl.BlockSpec((1, 1, DP), lambda h, i: (h, 0, 0)),
        ],
        out_specs=[
            pl.BlockSpec((BE, DP), lambda h, i: (h * NBE + i, 0)),
            pl.BlockSpec((BE, DP), lambda h, i: (h * NBE + i, 0)),
        ],
        out_shape=[
            jax.ShapeDtypeStruct((E, DP), jnp.float32),
            jax.ShapeDtypeStruct((E, DP), jnp.float32),
        ],
    )(ea, w1, b1, w2, b2)


def _update1_body(a_ref, x_ref, w1_ref, b1_ref, w2_ref, b2_ref, o_ref):
    s = a_ref[0] + a_ref[1] + x_ref[...]
    h = jnp.maximum(jnp.dot(s, w1_ref[...], preferred_element_type=jnp.float32)
                    + b1_ref[...], 0.0)
    f = jnp.dot(h, w2_ref[...], preferred_element_type=jnp.float32) + b2_ref[...]
    o_ref[...] = jnp.maximum(f, 0.0)


def _update1(agg, x_pad, w1, b1, w2, b2):
    return pl.pallas_call(
        _update1_body,
        grid=(NB,),
        in_specs=[
            pl.BlockSpec((2, BN, DP), lambda i: (0, i, 0)),
            pl.BlockSpec((BN, DP), lambda i: (i, 0)),
            pl.BlockSpec((DP, DP), lambda i: (0, 0)),
            pl.BlockSpec((1, DP), lambda i: (0, 0)),
            pl.BlockSpec((DP, DP), lambda i: (0, 0)),
            pl.BlockSpec((1, DP), lambda i: (0, 0)),
        ],
        out_specs=pl.BlockSpec((BN, DP), lambda i: (i, 0)),
        out_shape=jax.ShapeDtypeStruct((N, DP), jnp.float32),
    )(agg, x_pad, w1, b1, w2, b2)


def _update2_body(a_ref, x_ref, w1_ref, b1_ref, w2_ref, b2_ref, g_ref, acc_ref):
    i = pl.program_id(0)
    s = a_ref[0] + a_ref[1] + x_ref[...]
    h = jnp.maximum(jnp.dot(s, w1_ref[...], preferred_element_type=jnp.float32)
                    + b1_ref[...], 0.0)
    cs = jnp.sum(h, axis=0, keepdims=True)

    @pl.when(i == 0)
    def _():
        acc_ref[...] = cs

    @pl.when(i > 0)
    def _():
        acc_ref[...] = acc_ref[...] + cs

    @pl.when(i == NB - 1)
    def _():
        g_ref[...] = (jnp.dot(acc_ref[...], w2_ref[...],
                              preferred_element_type=jnp.float32)
                      + float(N) * b2_ref[...])


def _update2(agg, f1r, w1, b1, w2, b2):
    return pl.pallas_call(
        _update2_body,
        grid=(NB,),
        in_specs=[
            pl.BlockSpec((2, BN, DP), lambda i: (0, i, 0)),
            pl.BlockSpec((BN, DP), lambda i: (i, 0)),
            pl.BlockSpec((DP, H2), lambda i: (0, 0)),
            pl.BlockSpec((1, H2), lambda i: (0, 0)),
            pl.BlockSpec((H2, H2), lambda i: (0, 0)),
            pl.BlockSpec((1, H2), lambda i: (0, 0)),
        ],
        out_specs=pl.BlockSpec((1, H2), lambda i: (0, 0)),
        out_shape=jax.ShapeDtypeStruct((1, H2), jnp.float32),
        scratch_shapes=[pltpu.VMEM((1, H2), jnp.float32)],
    )(agg, f1r, w1, b1, w2, b2)


def _edge_body(ps_hbm, pd_hbm, ea_hbm, gsrc_hbm, gdst_hbm, sdst_hbm, out_hbm,
               gsrc_v, gdst_v, sdst_v,
               s0, d0, e0, s1, d1, e1,
               agg_sh, ga0, gb0, gc0, ga1, gb1, gc1, ss0, ss1):
    c = lax.axis_index("c")
    s = lax.axis_index("s")
    wid = c * NSUB + s
    base = wid * EPT

    pltpu.sync_copy(gsrc_hbm.at[pl.ds(base, EPT)], gsrc_v)
    pltpu.sync_copy(gdst_hbm.at[pl.ds(base, EPT)], gdst_v)
    pltpu.sync_copy(sdst_hbm.at[wid], sdst_v)

    zero = jnp.zeros((VL,), jnp.float32)

    with jax.named_scope("agg_zero"):
        def zrow(i, _):
            for j in range(DP // VL):
                s0[i, pl.ds(j * VL, VL)] = zero
            return 0

        lax.fori_loop(0, KC, zrow, 0)

        def zcopy(j, _):
            pltpu.sync_copy(s0.at[pl.ds(0, KC)],
                            agg_sh.at[pl.ds(s * RPS + j * KC, KC)])
            return 0

        lax.fori_loop(0, RPS // KC, zcopy, 0)
        plsc.subcore_barrier()

    def compute(bs, bd, be):
        def row(i, _):
            for j in range(DP // VL):
                sl = pl.ds(j * VL, VL)
                v = bs[i, sl] + bd[i, sl] + be[i, sl]
                bs[i, sl] = jnp.maximum(v, 0.0)
            return 0

        lax.fori_loop(0, K, row, 0)

    def do_chunk(ci, bs, bd, be, ga, gb, gc, ss):
        cpa = pltpu.async_copy(
            ps_hbm.at[gsrc_v.at[pl.ds(ci * K, K)]], bs, ga)
        cpb = pltpu.async_copy(
            pd_hbm.at[gdst_v.at[pl.ds(ci * K, K)]], bd, gb)
        cpc = pltpu.async_copy(
            ea_hbm.at[pl.ds(base + ci * K, K)], be, gc)
        return cpa, cpb, cpc

    with jax.named_scope("edge_loop"):
        def pair(p, _):
            c0 = 2 * p
            c1 = c0 + 1
            w0 = do_chunk(c0, s0, d0, e0, ga0, gb0, gc0, ss0)
            w1 = do_chunk(c1, s1, d1, e1, ga1, gb1, gc1, ss1)
            for w in w0:
                w.wait()
            compute(s0, d0, e0)
            sc0 = pltpu.async_copy(s0, agg_sh.at[sdst_v.at[c0]], ss0, add=True)
            for w in w1:
                w.wait()
            compute(s1, d1, e1)
            sc1 = pltpu.async_copy(s1, agg_sh.at[sdst_v.at[c1]], ss1, add=True)
            sc0.wait()
            sc1.wait()
            return 0

        lax.fori_loop(0, CH // 2, pair, 0)

        ct = CH - 1
        wt = do_chunk(ct, s0, d0, e0, ga0, gb0, gc0, ss0)
        for w in wt:
            w.wait()
        compute(s0, d0, e0)
        pltpu.async_copy(s0, agg_sh.at[sdst_v.at[ct]], ss0, add=True).wait()
        plsc.subcore_barrier()

    with jax.named_scope("agg_out"):
        def ocopy(j, _):
            off = s * RPS + j * KC
            pltpu.sync_copy(agg_sh.at[pl.ds(off, KC)], s0.at[pl.ds(0, KC)])
            pltpu.sync_copy(s0.at[pl.ds(0, KC)], out_hbm.at[c, pl.ds(off, KC)])
            return 0

        lax.fori_loop(0, RPS // KC, ocopy, 0)


@functools.cache
def _edge_agg_fn():
    mesh = plsc.VectorSubcoreMesh(core_axis_name="c", subcore_axis_name="s")
    return pl.kernel(
        _edge_body,
        out_type=jax.ShapeDtypeStruct((2, N, DP), jnp.float32),
        mesh=mesh,
        scratch_types=[
            pltpu.VMEM((EPT,), jnp.int32),
            pltpu.VMEM((EPT,), jnp.int32),
            pltpu.VMEM((CH, K), jnp.int32),
            pltpu.VMEM((K, DP), jnp.float32),
            pltpu.VMEM((K, DP), jnp.float32),
            pltpu.VMEM((K, DP), jnp.float32),
            pltpu.VMEM((K, DP), jnp.float32),
            pltpu.VMEM((K, DP), jnp.float32),
            pltpu.VMEM((K, DP), jnp.float32),
            pltpu.VMEM_SHARED((N, DP), jnp.float32),
            pltpu.SemaphoreType.DMA,
            pltpu.SemaphoreType.DMA,
            pltpu.SemaphoreType.DMA,
            pltpu.SemaphoreType.DMA,
            pltpu.SemaphoreType.DMA,
            pltpu.SemaphoreType.DMA,
            pltpu.SemaphoreType.DMA,
            pltpu.SemaphoreType.DMA,
        ],
        compiler_params=pltpu.CompilerParams(use_tc_tiling_on_sc=False),
    )


def _edge_agg(ps, pd, ea, gsrc, gdst, sdst):
    return _edge_agg_fn()(ps, pd, ea, gsrc, gdst, sdst)


def _conv_weights(lin_W, lin_b, lin2_W, lin2_b):
    ws = jnp.stack([_pad_w(lin_W[D + D:]), _pad_w(lin2_W[:D])])
    wd = jnp.stack([_pad_w(lin_W[:D]), _pad_w(lin2_W[D + D:])])
    wm = jnp.stack([lin_W[D:D + D], lin2_W[D:D + D]])
    wm = jnp.pad(wm, ((0, 0), (0, 0), (0, DP - D)))
    bm = jnp.stack([_pad_b(lin_b), _pad_b(lin2_b)])
    return ws, wd, wm, bm


def _two_graphs(g1, g2, w):
    (ws1, wd1, u1w1, u1b1, u1w2, u1b2,
     ws2, wd2, u2w1, u2b1, u2w2, u2b2) = w
    (x1, gsrc1, gdst1, sdst1, ea1_c1, ea1_c2) = g1
    (x2, gsrc2, gdst2, sdst2, ea2_c1, ea2_c2) = g2

    ps1, pd1 = _node_proj(x1, ws1, wd1)
    ps2, pd2 = _node_proj(x2, ws1, wd1)
    a1 = _edge_agg(ps1.reshape(2 * N, DP), pd1.reshape(2 * N, DP),
                   ea1_c1, gsrc1, gdst1, sdst1)
    a2 = _edge_agg(ps2.reshape(2 * N, DP), pd2.reshape(2 * N, DP),
                   ea2_c1, gsrc2, gdst2, sdst2)
    f1 = _update1(a1, x1, u1w1, u1b1, u1w2, u1b2)
    f2 = _update1(a2, x2, u1w1, u1b1, u1w2, u1b2)
    qs1, qd1 = _node_proj(f1, ws2, wd2)
    qs2, qd2 = _node_proj(f2, ws2, wd2)
    b1 = _edge_agg(qs1.reshape(2 * N, DP), qd1.reshape(2 * N, DP),
                   ea1_c2, gsrc1, gdst1, sdst1)
    b2 = _edge_agg(qs2.reshape(2 * N, DP), qd2.reshape(2 * N, DP),
                   ea2_c2, gsrc2, gdst2, sdst2)
    o1 = _update2(b1, f1, u2w1, u2b1, u2w2, u2b2)
    o2 = _update2(b2, f2, u2w1, u2b1, u2w2, u2b2)
    return o1, o2


def kernel(node_features_1, edge_index_1, edge_features_1,
           node_features_2, edge_index_2, edge_features_2,
           c1_lin_W, c1_lin_b, c1_lin2_W, c1_lin2_b,
           c1_mlp_W1, c1_mlp_b1, c1_mlp_W2, c1_mlp_b2,
           c2_lin_W, c2_lin_b, c2_lin2_W, c2_lin2_b,
           c2_mlp_W1, c2_mlp_b1, c2_mlp_W2, c2_mlp_b2):
    ws1, wd1, wm1, bm1 = _conv_weights(c1_lin_W, c1_lin_b, c1_lin2_W, c1_lin2_b)
    ws2, wd2, wm2, bm2 = _conv_weights(c2_lin_W, c2_lin_b, c2_lin2_W, c2_lin2_b)
    u1w1 = _pad_w(c1_mlp_W1)
    u1b1 = _pad_b(c1_mlp_b1)
    u1w2 = _pad_w(c1_mlp_W2)
    u1b2 = _pad_b(c1_mlp_b2)
    u2w1 = jnp.pad(c2_mlp_W1, ((0, DP - D), (0, 0)))
    u2b1 = c2_mlp_b1[None, :]
    u2w2 = c2_mlp_W2
    u2b2 = c2_mlp_b2[None, :]
    w = (ws1, wd1, u1w1, u1b1, u1w2, u1b2,
         ws2, wd2, u2w1, u2b1, u2w2, u2b2)

    halfmask = (jnp.arange(E, dtype=jnp.int32) >= HALF).astype(jnp.int32) * N

    def prep_graph(x, ei):
        x_pad = jnp.pad(x, ((0, 0), (0, DP - D)))
        src, dst = ei[0], ei[1]
        gsrc = src + halfmask
        gdst = dst + halfmask
        sdst = dst.reshape(NTILES, CH, K)
        return x_pad, gsrc, gdst, sdst

    x1, gsrc1, gdst1, sdst1 = prep_graph(node_features_1, edge_index_1)
    x2, gsrc2, gdst2, sdst2 = prep_graph(node_features_2, edge_index_2)

    ea1_c1, ea1_c2 = _ea_proj(edge_features_1, wm1, bm1, wm2, bm2)
    ea2_c1, ea2_c2 = _ea_proj(edge_features_2, wm1, bm1, wm2, bm2)

    o1, o2 = _two_graphs((x1, gsrc1, gdst1, sdst1, ea1_c1, ea1_c2),
                         (x2, gsrc2, gdst2, sdst2, ea2_c1, ea2_c2), w)
    return jnp.concatenate([o1, o2], axis=0)


# speedup vs baseline: 1.0693x; 1.0693x over previous
import functools

import jax
import jax.numpy as jnp
from jax import lax
from jax.experimental import pallas as pl
from jax.experimental.pallas import tpu as pltpu
from jax.experimental.pallas import tpu_sc as plsc

N = 10000
E = 160000
HALF = E // 2
D = 100
DP = 128
H2 = 64
NTILES = 32
NSUB = 16
EPT = E // NTILES
K = 40
CH = EPT // K
RPS = N // NSUB
KC = 25
BN = 400
NB = N // BN
BE = 640
NBE = HALF // BE
VL = 16


def _pad_w(w):
    r, c = w.shape
    return jnp.pad(w, ((0, DP - r), (0, DP - c)))


def _pad_b(b):
    return jnp.pad(b, (0, DP - b.shape[0]))[None, :]


def _proj_body(x_ref, ws_ref, wd_ref, ps_ref, pd_ref):
    x = x_ref[...]
    ps_ref[0] = jnp.dot(x, ws_ref[0], preferred_element_type=jnp.float32)
    pd_ref[0] = jnp.dot(x, wd_ref[0], preferred_element_type=jnp.float32)


def _node_proj(x_pad, ws, wd):
    return pl.pallas_call(
        _proj_body,
        grid=(2, NB),
        in_specs=[
            pl.BlockSpec((BN, DP), lambda h, i: (i, 0)),
            pl.BlockSpec((1, DP, DP), lambda h, i: (h, 0, 0)),
            pl.BlockSpec((1, DP, DP), lambda h, i: (h, 0, 0)),
        ],
        out_specs=[
            pl.BlockSpec((1, BN, DP), lambda h, i: (h, i, 0)),
            pl.BlockSpec((1, BN, DP), lambda h, i: (h, i, 0)),
        ],
        out_shape=[
            jax.ShapeDtypeStruct((2, N, DP), jnp.float32),
            jax.ShapeDtypeStruct((2, N, DP), jnp.float32),
        ],
    )(x_pad, ws, wd)


def _ea_body(eat_ref, w1_ref, b1_ref, w2_ref, b2_ref, o1_ref, o2_ref):
    eat = eat_ref[...]
    dn = (((0,), (0,)), ((), ()))
    o1_ref[...] = lax.dot_general(eat, w1_ref[0], dn,
                                  preferred_element_type=jnp.float32) + b1_ref[0]
    o2_ref[...] = lax.dot_general(eat, w2_ref[0], dn,
                                  preferred_element_type=jnp.float32) + b2_ref[0]


def _ea_proj(ea_t, w1, b1, w2, b2):
    return pl.pallas_call(
        _ea_body,
        grid=(2, NBE),
        in_specs=[
            pl.BlockSpec((D, BE), lambda h, i: (0, h * NBE + i)),
            pl.BlockSpec((1, D, DP), lambda h, i: (h, 0, 0)),
            pl.BlockSpec((1, 1, DP), lambda h, i: (h, 0, 0)),
            pl.BlockSpec((1, D, DP), lambda h, i: (h, 0, 0)),
            pl.BlockSpec((1, 1, DP), lambda h, i: (h, 0, 0)),
        ],
        out_specs=[
            pl.BlockSpec((BE, DP), lambda h, i: (h * NBE + i, 0)),
            pl.BlockSpec((BE, DP), lambda h, i: (h * NBE + i, 0)),
        ],
        out_shape=[
            jax.ShapeDtypeStruct((E, DP), jnp.float32),
            jax.ShapeDtypeStruct((E, DP), jnp.float32),
        ],
    )(ea_t, w1, b1, w2, b2)


def _update1_body(a_ref, x_ref, w1_ref, b1_ref, w2_ref, b2_ref, o_ref):
    s = a_ref[0] + a_ref[1] + x_ref[...]
    h = jnp.maximum(jnp.dot(s, w1_ref[...], preferred_element_type=jnp.float32)
                    + b1_ref[...], 0.0)
    f = jnp.dot(h, w2_ref[...], preferred_element_type=jnp.float32) + b2_ref[...]
    o_ref[...] = jnp.maximum(f, 0.0)


def _update1(agg, x_pad, w1, b1, w2, b2):
    return pl.pallas_call(
        _update1_body,
        grid=(NB,),
        in_specs=[
            pl.BlockSpec((2, BN, DP), lambda i: (0, i, 0)),
            pl.BlockSpec((BN, DP), lambda i: (i, 0)),
            pl.BlockSpec((DP, DP), lambda i: (0, 0)),
            pl.BlockSpec((1, DP), lambda i: (0, 0)),
            pl.BlockSpec((DP, DP), lambda i: (0, 0)),
            pl.BlockSpec((1, DP), lambda i: (0, 0)),
        ],
        out_specs=pl.BlockSpec((BN, DP), lambda i: (i, 0)),
        out_shape=jax.ShapeDtypeStruct((N, DP), jnp.float32),
    )(agg, x_pad, w1, b1, w2, b2)


def _update2_body(a_ref, x_ref, w1_ref, b1_ref, w2_ref, b2_ref, g_ref, acc_ref):
    i = pl.program_id(0)
    s = a_ref[0] + a_ref[1] + x_ref[...]
    h = jnp.maximum(jnp.dot(s, w1_ref[...], preferred_element_type=jnp.float32)
                    + b1_ref[...], 0.0)
    cs = jnp.sum(h, axis=0, keepdims=True)

    @pl.when(i == 0)
    def _():
        acc_ref[...] = cs

    @pl.when(i > 0)
    def _():
        acc_ref[...] = acc_ref[...] + cs

    @pl.when(i == NB - 1)
    def _():
        g_ref[...] = (jnp.dot(acc_ref[...], w2_ref[...],
                              preferred_element_type=jnp.float32)
                      + float(N) * b2_ref[...])


def _update2(agg, f1r, w1, b1, w2, b2):
    return pl.pallas_call(
        _update2_body,
        grid=(NB,),
        in_specs=[
            pl.BlockSpec((2, BN, DP), lambda i: (0, i, 0)),
            pl.BlockSpec((BN, DP), lambda i: (i, 0)),
            pl.BlockSpec((DP, H2), lambda i: (0, 0)),
            pl.BlockSpec((1, H2), lambda i: (0, 0)),
            pl.BlockSpec((H2, H2), lambda i: (0, 0)),
            pl.BlockSpec((1, H2), lambda i: (0, 0)),
        ],
        out_specs=pl.BlockSpec((1, H2), lambda i: (0, 0)),
        out_shape=jax.ShapeDtypeStruct((1, H2), jnp.float32),
        scratch_shapes=[pltpu.VMEM((1, H2), jnp.float32)],
    )(agg, f1r, w1, b1, w2, b2)


def _edge_body(ps_hbm, pd_hbm, ea_hbm, gsrc_hbm, gdst_hbm, sdst_hbm, out_hbm,
               gsrc_v, gdst_v, sdst_v,
               s0, d0, e0, s1, d1, e1,
               agg_sh, ga0, gb0, gc0, ga1, gb1, gc1, ss0, ss1):
    c = lax.axis_index("c")
    s = lax.axis_index("s")
    wid = c * NSUB + s
    base = wid * EPT

    pltpu.sync_copy(gsrc_hbm.at[pl.ds(base, EPT)], gsrc_v)
    pltpu.sync_copy(gdst_hbm.at[pl.ds(base, EPT)], gdst_v)
    pltpu.sync_copy(sdst_hbm.at[wid], sdst_v)

    zero = jnp.zeros((VL,), jnp.float32)

    with jax.named_scope("agg_zero"):
        def zrow(i, _):
            for j in range(DP // VL):
                s0[i, pl.ds(j * VL, VL)] = zero
            return 0

        lax.fori_loop(0, KC, zrow, 0)

        def zcopy(j, _):
            pltpu.sync_copy(s0.at[pl.ds(0, KC)],
                            agg_sh.at[pl.ds(s * RPS + j * KC, KC)])
            return 0

        lax.fori_loop(0, RPS // KC, zcopy, 0)
        plsc.subcore_barrier()

    def compute(bs, bd, be):
        def row(i, _):
            for j in range(DP // VL):
                sl = pl.ds(j * VL, VL)
                v = bs[i, sl] + bd[i, sl] + be[i, sl]
                bs[i, sl] = jnp.maximum(v, 0.0)
            return 0

        lax.fori_loop(0, K, row, 0)

    def do_chunk(ci, bs, bd, be, ga, gb, gc, ss):
        cpa = pltpu.async_copy(
            ps_hbm.at[gsrc_v.at[pl.ds(ci * K, K)]], bs, ga)
        cpb = pltpu.async_copy(
            pd_hbm.at[gdst_v.at[pl.ds(ci * K, K)]], bd, gb)
        cpc = pltpu.async_copy(
            ea_hbm.at[pl.ds(base + ci * K, K)], be, gc)
        return cpa, cpb, cpc

    with jax.named_scope("edge_loop"):
        def pair(p, _):
            c0 = 2 * p
            c1 = c0 + 1
            w0 = do_chunk(c0, s0, d0, e0, ga0, gb0, gc0, ss0)
            w1 = do_chunk(c1, s1, d1, e1, ga1, gb1, gc1, ss1)
            for w in w0:
                w.wait()
            compute(s0, d0, e0)
            sc0 = pltpu.async_copy(s0, agg_sh.at[sdst_v.at[c0]], ss0, add=True)
            for w in w1:
                w.wait()
            compute(s1, d1, e1)
            sc1 = pltpu.async_copy(s1, agg_sh.at[sdst_v.at[c1]], ss1, add=True)
            sc0.wait()
            sc1.wait()
            return 0

        lax.fori_loop(0, CH // 2, pair, 0)

        ct = CH - 1
        wt = do_chunk(ct, s0, d0, e0, ga0, gb0, gc0, ss0)
        for w in wt:
            w.wait()
        compute(s0, d0, e0)
        pltpu.async_copy(s0, agg_sh.at[sdst_v.at[ct]], ss0, add=True).wait()
        plsc.subcore_barrier()

    with jax.named_scope("agg_out"):
        def ocopy(j, _):
            off = s * RPS + j * KC
            pltpu.sync_copy(agg_sh.at[pl.ds(off, KC)], s0.at[pl.ds(0, KC)])
            pltpu.sync_copy(s0.at[pl.ds(0, KC)], out_hbm.at[c, pl.ds(off, KC)])
            return 0

        lax.fori_loop(0, RPS // KC, ocopy, 0)


@functools.cache
def _edge_agg_fn():
    mesh = plsc.VectorSubcoreMesh(core_axis_name="c", subcore_axis_name="s")
    return pl.kernel(
        _edge_body,
        out_type=jax.ShapeDtypeStruct((2, N, DP), jnp.float32),
        mesh=mesh,
        scratch_types=[
            pltpu.VMEM((EPT,), jnp.int32),
            pltpu.VMEM((EPT,), jnp.int32),
            pltpu.VMEM((CH, K), jnp.int32),
            pltpu.VMEM((K, DP), jnp.float32),
            pltpu.VMEM((K, DP), jnp.float32),
            pltpu.VMEM((K, DP), jnp.float32),
            pltpu.VMEM((K, DP), jnp.float32),
            pltpu.VMEM((K, DP), jnp.float32),
            pltpu.VMEM((K, DP), jnp.float32),
            pltpu.VMEM_SHARED((N, DP), jnp.float32),
            pltpu.SemaphoreType.DMA,
            pltpu.SemaphoreType.DMA,
            pltpu.SemaphoreType.DMA,
            pltpu.SemaphoreType.DMA,
            pltpu.SemaphoreType.DMA,
            pltpu.SemaphoreType.DMA,
            pltpu.SemaphoreType.DMA,
            pltpu.SemaphoreType.DMA,
        ],
        compiler_params=pltpu.CompilerParams(use_tc_tiling_on_sc=False),
    )


def _edge_agg(ps, pd, ea, gsrc, gdst, sdst):
    return _edge_agg_fn()(ps, pd, ea, gsrc, gdst, sdst)


def _conv_weights(lin_W, lin_b, lin2_W, lin2_b):
    ws = jnp.stack([_pad_w(lin_W[D + D:]), _pad_w(lin2_W[:D])])
    wd = jnp.stack([_pad_w(lin_W[:D]), _pad_w(lin2_W[D + D:])])
    wm = jnp.stack([lin_W[D:D + D], lin2_W[D:D + D]])
    wm = jnp.pad(wm, ((0, 0), (0, 0), (0, DP - D)))
    bm = jnp.stack([_pad_b(lin_b), _pad_b(lin2_b)])
    return ws, wd, wm, bm


def _two_graphs(g1, g2, w):
    (ws1, wd1, u1w1, u1b1, u1w2, u1b2,
     ws2, wd2, u2w1, u2b1, u2w2, u2b2) = w
    (x1, gsrc1, gdst1, sdst1, ea1_c1, ea1_c2) = g1
    (x2, gsrc2, gdst2, sdst2, ea2_c1, ea2_c2) = g2

    ps1, pd1 = _node_proj(x1, ws1, wd1)
    ps2, pd2 = _node_proj(x2, ws1, wd1)
    a1 = _edge_agg(ps1.reshape(2 * N, DP), pd1.reshape(2 * N, DP),
                   ea1_c1, gsrc1, gdst1, sdst1)
    a2 = _edge_agg(ps2.reshape(2 * N, DP), pd2.reshape(2 * N, DP),
                   ea2_c1, gsrc2, gdst2, sdst2)
    f1 = _update1(a1, x1, u1w1, u1b1, u1w2, u1b2)
    f2 = _update1(a2, x2, u1w1, u1b1, u1w2, u1b2)
    qs1, qd1 = _node_proj(f1, ws2, wd2)
    qs2, qd2 = _node_proj(f2, ws2, wd2)
    b1 = _edge_agg(qs1.reshape(2 * N, DP), qd1.reshape(2 * N, DP),
                   ea1_c2, gsrc1, gdst1, sdst1)
    b2 = _edge_agg(qs2.reshape(2 * N, DP), qd2.reshape(2 * N, DP),
                   ea2_c2, gsrc2, gdst2, sdst2)
    o1 = _update2(b1, f1, u2w1, u2b1, u2w2, u2b2)
    o2 = _update2(b2, f2, u2w1, u2b1, u2w2, u2b2)
    return o1, o2


def kernel(node_features_1, edge_index_1, edge_features_1,
           node_features_2, edge_index_2, edge_features_2,
           c1_lin_W, c1_lin_b, c1_lin2_W, c1_lin2_b,
           c1_mlp_W1, c1_mlp_b1, c1_mlp_W2, c1_mlp_b2,
           c2_lin_W, c2_lin_b, c2_lin2_W, c2_lin2_b,
           c2_mlp_W1, c2_mlp_b1, c2_mlp_W2, c2_mlp_b2):
    ws1, wd1, wm1, bm1 = _conv_weights(c1_lin_W, c1_lin_b, c1_lin2_W, c1_lin2_b)
    ws2, wd2, wm2, bm2 = _conv_weights(c2_lin_W, c2_lin_b, c2_lin2_W, c2_lin2_b)
    u1w1 = _pad_w(c1_mlp_W1)
    u1b1 = _pad_b(c1_mlp_b1)
    u1w2 = _pad_w(c1_mlp_W2)
    u1b2 = _pad_b(c1_mlp_b2)
    u2w1 = jnp.pad(c2_mlp_W1, ((0, DP - D), (0, 0)))
    u2b1 = c2_mlp_b1[None, :]
    u2w2 = c2_mlp_W2
    u2b2 = c2_mlp_b2[None, :]
    w = (ws1, wd1, u1w1, u1b1, u1w2, u1b2,
         ws2, wd2, u2w1, u2b1, u2w2, u2b2)

    halfmask = (jnp.arange(E, dtype=jnp.int32) >= HALF).astype(jnp.int32) * N

    def prep_graph(x, ei):
        x_pad = jnp.pad(x, ((0, 0), (0, DP - D)))
        src, dst = ei[0], ei[1]
        gsrc = src + halfmask
        gdst = dst + halfmask
        sdst = dst.reshape(NTILES, CH, K)
        return x_pad, gsrc, gdst, sdst

    x1, gsrc1, gdst1, sdst1 = prep_graph(node_features_1, edge_index_1)
    x2, gsrc2, gdst2, sdst2 = prep_graph(node_features_2, edge_index_2)

    ea1_c1, ea1_c2 = _ea_proj(edge_features_1.T, wm1, bm1, wm2, bm2)
    ea2_c1, ea2_c2 = _ea_proj(edge_features_2.T, wm1, bm1, wm2, bm2)

    o1, o2 = _two_graphs((x1, gsrc1, gdst1, sdst1, ea1_c1, ea1_c2),
                         (x2, gsrc2, gdst2, sdst2, ea2_c1, ea2_c2), w)
    return jnp.concatenate([o1, o2], axis=0)
